# baseline (speedup 1.0000x reference)
import functools

import jax
import jax.numpy as jnp
from jax import lax
from jax.experimental import pallas as pl
from jax.experimental.pallas import tpu as pltpu
from jax.experimental.pallas import tpu_sc as plsc

DIM = 64
LANES = 16
CHUNK = 128
BLOCKS = CHUNK // LANES
SUPER = 8192
QUART = SUPER // 4
QBITS = QUART.bit_length() - 1
PITCH = LANES + 1
NBUF = 2


def _sqrt(s):
    bits = lax.bitcast_convert_type(s, jnp.int32)
    y = lax.bitcast_convert_type(
        jnp.int32(0x5F3759DF) - lax.shift_right_logical(bits, 1), jnp.float32)
    for _ in range(3):
        y = y * (1.5 - 0.5 * s * y * y)
    return s * y


def _repack_entity(entity_t):
    dim, n_ent = entity_t.shape
    nblk = (n_ent + SUPER - 1) // SUPER

    def body(in_ref, out_ref):
        x = in_ref[...]
        ii = lax.broadcasted_iota(jnp.int32, (dim, dim), 0)
        jj = lax.broadcasted_iota(jnp.int32, (dim, dim), 1)
        ident = jnp.where(ii == jj, 1.0, 0.0).astype(jnp.float32)
        dn = (((0,), (0,)), ((), ()))

        def tq(lo):
            t = lax.dot_general(x[:, lo:lo + QUART], ident,
                                dimension_numbers=dn,
                                preferred_element_type=jnp.float32)
            b16 = lax.bitcast_convert_type(t.astype(jnp.bfloat16), jnp.uint16)
            return b16.astype(jnp.int32)

        a, b, c, d = tq(0), tq(QUART), tq(2 * QUART), tq(3 * QUART)
        out_ref[...] = jnp.concatenate(
            [a | lax.shift_left(c, 16), b | lax.shift_left(d, 16)], axis=1)

    return pl.pallas_call(
        body,
        grid=(nblk,),
        in_specs=[pl.BlockSpec((dim, SUPER), lambda i: (0, i))],
        out_specs=pl.BlockSpec((QUART, 2 * dim), lambda i: (i, 0)),
        out_shape=jax.ShapeDtypeStruct((nblk * QUART, 2 * dim), jnp.int32),
    )(entity_t)


def _sc_kernel(batch, n_workers):
    rows_per_worker = batch // n_workers
    n_chunks = rows_per_worker // CHUNK
    mesh = plsc.VectorSubcoreMesh(core_axis_name="c", subcore_axis_name="s")

    @functools.partial(
        pl.kernel,
        mesh=mesh,
        compiler_params=pltpu.CompilerParams(
            needs_layout_passes=False, use_tc_tiling_on_sc=True),
        out_type=jax.ShapeDtypeStruct((batch,), jnp.float32),
        scratch_types=[
            pltpu.VMEM((rows_per_worker,), jnp.int32),
            pltpu.VMEM((rows_per_worker,), jnp.int32),
            pltpu.VMEM((rows_per_worker,), jnp.int32),
            pltpu.VMEM((rows_per_worker,), jnp.int32),
            pltpu.VMEM((rows_per_worker,), jnp.int32),
            pltpu.VMEM((rows_per_worker,), jnp.int32),
            pltpu.VMEM((rows_per_worker,), jnp.int32),
            pltpu.VMEM((rows_per_worker,), jnp.int32),
            pltpu.VMEM((NBUF, CHUNK, 2 * DIM), jnp.int32),
            pltpu.VMEM((NBUF, CHUNK, 2 * DIM), jnp.int32),
            pltpu.VMEM((500 * DIM,), jnp.int32),
            pltpu.VMEM((LANES * PITCH,), jnp.float32),
            pltpu.VMEM((rows_per_worker,), jnp.float32),
            pltpu.SemaphoreType.DMA,
            pltpu.SemaphoreType.DMA,
            pltpu.SemaphoreType.DMA,
        ],
    )
    def k(heads, relations, tails, entity2, relation2, out,
          hq, rq, tq, hc, tc, hs, rs, ts, hrows, trows, rel_local,
          st, outc, sem0, sem1, osem):
        n_cores = 2
        wid = lax.axis_index("s") * n_cores + lax.axis_index("c")
        base = wid * rows_per_worker
        lane_iota = lax.iota(jnp.int32, LANES)
        sems = [sem0, sem1]

        pltpu.sync_copy(relation2, rel_local)
        pltpu.sync_copy(heads.at[pl.ds(base, rows_per_worker)], hq)
        pltpu.sync_copy(relations.at[pl.ds(base, rows_per_worker)], rq)
        pltpu.sync_copy(tails.at[pl.ds(base, rows_per_worker)], tq)

        def remap_body(i, _):
            sl = pl.ds(i * LANES, LANES)
            hv = hq[sl]
            tv = tq[sl]
            rv = rq[sl]
            hc[sl] = (lax.shift_right_logical(hv, QBITS) & 1) * DIM
            tc[sl] = (lax.shift_right_logical(tv, QBITS) & 1) * DIM
            hs[sl] = 16 - ((lax.shift_right_logical(hv, QBITS + 1) & 1) * 16)
            ts[sl] = 16 - ((lax.shift_right_logical(tv, QBITS + 1) & 1) * 16)
            rs[sl] = 16 - ((rv & 1) * 16)
            hq[sl] = ((lax.shift_right_logical(hv, QBITS + 2) * QUART)
                      | (hv & (QUART - 1)))
            tq[sl] = ((lax.shift_right_logical(tv, QBITS + 2) * QUART)
                      | (tv & (QUART - 1)))
            rq[sl] = lax.shift_right_logical(rv, 1) * DIM
            return 0

        lax.fori_loop(0, rows_per_worker // LANES, remap_body, 0)

        def stage(c, slot):
            csl = pl.ds(c * CHUNK, CHUNK)
            sem = sems[slot]
            return (pltpu.async_copy(entity2.at[hq.at[csl]],
                                     hrows.at[slot], sem),
                    pltpu.async_copy(entity2.at[tq.at[csl]],
                                     trows.at[slot], sem))

        def compute(c, slot):
            hb, tb = hrows.at[slot], trows.at[slot]
            c0 = c * CHUNK

            def blk_body(b, _):
                r0 = b * LANES
                bsl = pl.ds(c0 + r0, LANES)
                hcv = hc[bsl]
                rqv = rq[bsl]
                tcv = tc[bsl]
                hsv = hs[bsl]
                rsv = rs[bsl]
                tsv = ts[bsl]
                mask = jnp.int32(-65536)
                for u in range(LANES):
                    r = r0 + u
                    ho = hcv[u]
                    ro = rqv[u]
                    to = tcv[u]
                    hsh = jnp.full((LANES,), hsv[u], jnp.int32)
                    rsh = jnp.full((LANES,), rsv[u], jnp.int32)
                    tsh = jnp.full((LANES,), tsv[u], jnp.int32)
                    s = jnp.zeros((LANES,), jnp.float32)
                    for kk in range(DIM // LANES):
                        hx = hb[r, pl.ds(ho + kk * LANES, LANES)]
                        rx = rel_local[pl.ds(ro + kk * LANES, LANES)]
                        tx = tb[r, pl.ds(to + kk * LANES, LANES)]
                        hval = lax.bitcast_convert_type(
                            lax.shift_left(hx, hsh) & mask, jnp.float32)
                        rval = lax.bitcast_convert_type(
                            lax.shift_left(rx, rsh) & mask, jnp.float32)
                        tval = lax.bitcast_convert_type(
                            lax.shift_left(tx, tsh) & mask, jnp.float32)
                        d = hval + rval - tval
                        s = s + d * d
                    plsc.store_scatter(st, [lane_iota * PITCH + u], s)
                acc = st[pl.ds(0, LANES)]
                for j in range(1, LANES):
                    acc = acc + st[pl.ds(j * PITCH, LANES)]
                outc[pl.ds(c0 + r0, LANES)] = _sqrt(acc)
                return 0

            lax.fori_loop(0, BLOCKS, blk_body, 0)
            return pltpu.async_copy(
                outc.at[pl.ds(c0, CHUNK)],
                out.at[pl.ds(base + c0, CHUNK)], osem)

        pending = stage(0, 0)
        ocopies = []
        for c in range(n_chunks):
            nxt = stage(c + 1, (c + 1) % NBUF) if c + 1 < n_chunks else None
            for cp in pending:
                cp.wait()
            ocopies.append(compute(c, c % NBUF))
            pending = nxt
        for oc in ocopies:
            oc.wait()

    return k


def kernel(heads, relations, tails, entity_emb, relation_emb):
    batch = heads.shape[0]
    entity2 = _repack_entity(entity_emb.T)
    r16 = relation_emb.astype(jnp.bfloat16)
    lo = lax.bitcast_convert_type(r16[0::2, :], jnp.uint16).astype(jnp.int32)
    hi = lax.bitcast_convert_type(r16[1::2, :], jnp.uint16).astype(jnp.int32)
    relation2 = (lo | lax.shift_left(hi, 16)).reshape(-1)
    k = _sc_kernel(batch, 32)
    return k(heads.astype(jnp.int32), relations.astype(jnp.int32),
             tails.astype(jnp.int32), entity2, relation2)


# speedup vs baseline: 1.0423x; 1.0423x over previous
import functools

import jax
import jax.numpy as jnp
from jax import lax
from jax.experimental import pallas as pl
from jax.experimental.pallas import tpu as pltpu
from jax.experimental.pallas import tpu_sc as plsc

DIM = 64
LANES = 16
CHUNK = 128
BLOCKS = CHUNK // LANES
SUPER = 8192
QUART = SUPER // 4
QBITS = QUART.bit_length() - 1
PITCH = LANES + 1
NBUF = 2


def _sqrt(s):
    bits = lax.bitcast_convert_type(s, jnp.int32)
    y = lax.bitcast_convert_type(
        jnp.int32(0x5F3759DF) - lax.shift_right_logical(bits, 1), jnp.float32)
    for _ in range(3):
        y = y * (1.5 - 0.5 * s * y * y)
    return s * y


def _repack_entity(entity_t):
    dim, n_ent = entity_t.shape
    nblk = (n_ent + SUPER - 1) // SUPER

    def body(in_ref, out_ref):
        x = in_ref[...]
        ii = lax.broadcasted_iota(jnp.int32, (dim, dim), 0)
        jj = lax.broadcasted_iota(jnp.int32, (dim, dim), 1)
        ident = jnp.where(ii == jj, 1.0, 0.0).astype(jnp.float32)
        dn = (((0,), (0,)), ((), ()))

        def tq(lo):
            t = lax.dot_general(x[:, lo:lo + QUART], ident,
                                dimension_numbers=dn,
                                preferred_element_type=jnp.float32)
            b16 = lax.bitcast_convert_type(t.astype(jnp.bfloat16), jnp.uint16)
            return b16.astype(jnp.int32)

        a, b, c, d = tq(0), tq(QUART), tq(2 * QUART), tq(3 * QUART)
        out_ref[...] = jnp.concatenate(
            [a | lax.shift_left(c, 16), b | lax.shift_left(d, 16)], axis=1)

    return pl.pallas_call(
        body,
        grid=(nblk,),
        in_specs=[pl.BlockSpec((dim, SUPER), lambda i: (0, i))],
        out_specs=pl.BlockSpec((QUART, 2 * dim), lambda i: (i, 0)),
        out_shape=jax.ShapeDtypeStruct((nblk * QUART, 2 * dim), jnp.int32),
    )(entity_t)


def _sc_kernel(batch, n_workers):
    rows_per_worker = batch // n_workers
    n_chunks = rows_per_worker // CHUNK
    mesh = plsc.VectorSubcoreMesh(core_axis_name="c", subcore_axis_name="s")

    @functools.partial(
        pl.kernel,
        mesh=mesh,
        compiler_params=pltpu.CompilerParams(
            needs_layout_passes=False, use_tc_tiling_on_sc=True),
        out_type=jax.ShapeDtypeStruct((batch,), jnp.float32),
        scratch_types=[
            pltpu.VMEM((rows_per_worker,), jnp.int32),
            pltpu.VMEM((rows_per_worker,), jnp.int32),
            pltpu.VMEM((rows_per_worker,), jnp.int32),
            pltpu.VMEM((rows_per_worker,), jnp.int32),
            pltpu.VMEM((rows_per_worker,), jnp.int32),
            pltpu.VMEM((rows_per_worker,), jnp.int32),
            pltpu.VMEM((rows_per_worker,), jnp.int32),
            pltpu.VMEM((rows_per_worker,), jnp.int32),
            pltpu.VMEM((NBUF, CHUNK, 2 * DIM), jnp.int32),
            pltpu.VMEM((NBUF, CHUNK, 2 * DIM), jnp.float32),
            pltpu.VMEM((NBUF, CHUNK, 2 * DIM), jnp.int32),
            pltpu.VMEM((LANES * PITCH,), jnp.float32),
            pltpu.VMEM((rows_per_worker,), jnp.float32),
            pltpu.SemaphoreType.DMA,
            pltpu.SemaphoreType.DMA,
            pltpu.SemaphoreType.DMA,
        ],
    )
    def k(heads, relations, tails, entity2, relation2, out,
          hq, rq, tq, hc, rc, tc, hs, ts, hrows, rrows, trows, st, outc,
          sem0, sem1, osem):
        n_cores = 2
        wid = lax.axis_index("s") * n_cores + lax.axis_index("c")
        base = wid * rows_per_worker
        lane_iota = lax.iota(jnp.int32, LANES)
        sems = [sem0, sem1]

        pltpu.sync_copy(heads.at[pl.ds(base, rows_per_worker)], hq)
        pltpu.sync_copy(relations.at[pl.ds(base, rows_per_worker)], rq)
        pltpu.sync_copy(tails.at[pl.ds(base, rows_per_worker)], tq)

        def remap_body(i, _):
            sl = pl.ds(i * LANES, LANES)
            hv = hq[sl]
            tv = tq[sl]
            rv = rq[sl]
            hc[sl] = (lax.shift_right_logical(hv, QBITS) & 1) * DIM
            tc[sl] = (lax.shift_right_logical(tv, QBITS) & 1) * DIM
            rc[sl] = (rv & 1) * DIM
            hs[sl] = 16 - ((lax.shift_right_logical(hv, QBITS + 1) & 1) * 16)
            ts[sl] = 16 - ((lax.shift_right_logical(tv, QBITS + 1) & 1) * 16)
            hq[sl] = ((lax.shift_right_logical(hv, QBITS + 2) * QUART)
                      | (hv & (QUART - 1)))
            tq[sl] = ((lax.shift_right_logical(tv, QBITS + 2) * QUART)
                      | (tv & (QUART - 1)))
            rq[sl] = lax.shift_right_logical(rv, 1)
            return 0

        lax.fori_loop(0, rows_per_worker // LANES, remap_body, 0)

        def stage(c, slot):
            csl = pl.ds(c * CHUNK, CHUNK)
            sem = sems[slot]
            return (pltpu.async_copy(entity2.at[hq.at[csl]],
                                     hrows.at[slot], sem),
                    pltpu.async_copy(relation2.at[rq.at[csl]],
                                     rrows.at[slot], sem),
                    pltpu.async_copy(entity2.at[tq.at[csl]],
                                     trows.at[slot], sem))

        def compute(c, slot):
            hb, rb, tb = hrows.at[slot], rrows.at[slot], trows.at[slot]
            c0 = c * CHUNK

            def blk_body(b, _):
                r0 = b * LANES
                bsl = pl.ds(c0 + r0, LANES)
                hcv = hc[bsl]
                rcv = rc[bsl]
                tcv = tc[bsl]
                hsv = hs[bsl]
                tsv = ts[bsl]
                mask = jnp.int32(-65536)
                for u in range(LANES):
                    r = r0 + u
                    ho = hcv[u]
                    ro = rcv[u]
                    to = tcv[u]
                    hsh = jnp.full((LANES,), hsv[u], jnp.int32)
                    tsh = jnp.full((LANES,), tsv[u], jnp.int32)
                    s = jnp.zeros((LANES,), jnp.float32)
                    for kk in range(DIM // LANES):
                        hx = hb[r, pl.ds(ho + kk * LANES, LANES)]
                        tx = tb[r, pl.ds(to + kk * LANES, LANES)]
                        hval = lax.bitcast_convert_type(
                            lax.shift_left(hx, hsh) & mask, jnp.float32)
                        tval = lax.bitcast_convert_type(
                            lax.shift_left(tx, tsh) & mask, jnp.float32)
                        d = hval + rb[r, pl.ds(ro + kk * LANES, LANES)] - tval
                        s = s + d * d
                    plsc.store_scatter(st, [lane_iota * PITCH + u], s)
                acc = st[pl.ds(0, LANES)]
                for j in range(1, LANES):
                    acc = acc + st[pl.ds(j * PITCH, LANES)]
                outc[pl.ds(c0 + r0, LANES)] = _sqrt(acc)
                return 0

            lax.fori_loop(0, BLOCKS, blk_body, 0)
            return pltpu.async_copy(
                outc.at[pl.ds(c0, CHUNK)],
                out.at[pl.ds(base + c0, CHUNK)], osem)

        pending = stage(0, 0)
        ocopies = []
        for c in range(n_chunks):
            nxt = stage(c + 1, (c + 1) % NBUF) if c + 1 < n_chunks else None
            for cp in pending:
                cp.wait()
            ocopies.append(compute(c, c % NBUF))
            pending = nxt
        for oc in ocopies:
            oc.wait()

    return k


def kernel(heads, relations, tails, entity_emb, relation_emb):
    batch = heads.shape[0]
    entity2 = _repack_entity(entity_emb.T)
    relation2 = relation_emb.reshape(relation_emb.shape[0] // 2, 2 * DIM)
    k = _sc_kernel(batch, 32)
    return k(heads.astype(jnp.int32), relations.astype(jnp.int32),
             tails.astype(jnp.int32), entity2, relation2)
